# baseline (speedup 1.0000x reference)
import functools

import jax
import jax.numpy as jnp
from jax import lax
from jax.experimental import pallas as pl
from jax.experimental.pallas import tpu as pltpu
from jax.experimental.pallas import tpu_sc as plsc

_D = 32
_L = 16
_NC = 2
_NS = 16
_NW = _NC * _NS
_NBUF = 14


@functools.lru_cache(maxsize=None)
def _build(batch, num_rows):
    bpw = batch // _NW
    last_win = ((num_rows - 1) // 128) * 128
    mesh = plsc.VectorSubcoreMesh(core_axis_name="c", subcore_axis_name="s")

    @functools.partial(
        pl.kernel,
        mesh=mesh,
        compiler_params=pltpu.CompilerParams(
            needs_layout_passes=False, disable_bounds_checks=True),
        out_type=jax.ShapeDtypeStruct((batch,), jnp.float32),
        scratch_types=[
            pltpu.VMEM((bpw,), jnp.int32),
            pltpu.VMEM((bpw,), jnp.int32),
            pltpu.VMEM((bpw,), jnp.int32),
            pltpu.VMEM((bpw,), jnp.int32),
            pltpu.VMEM((_NBUF, _D, 128), jnp.float32),
            pltpu.VMEM((_NBUF, _D, 128), jnp.float32),
            pltpu.VMEM((bpw,), jnp.float32),
            pltpu.SemaphoreType.DMA((_NBUF,)),
        ],
    )
    def _k(uidx_hbm, iidx_hbm, utab_t, itab_t, out_hbm,
           uws_v, iws_v, ul_v, il_v, uwin, iwin, out_v, sems):
        wid = lax.axis_index("s") * _NC + lax.axis_index("c")
        base = wid * bpw
        lane16 = lax.iota(jnp.int32, _L)

        pltpu.sync_copy(uidx_hbm.at[pl.ds(base, bpw)], ul_v)
        pltpu.sync_copy(iidx_hbm.at[pl.ds(base, bpw)], il_v)
        for c in range(bpw // _L):
            u = ul_v[pl.ds(c * _L, _L)]
            ws = jnp.minimum((u >> 7) << 7, last_win)
            uws_v[pl.ds(c * _L, _L)] = ws
            ul_v[pl.ds(c * _L, _L)] = u - ws
            v = il_v[pl.ds(c * _L, _L)]
            ws = jnp.minimum((v >> 7) << 7, last_win)
            iws_v[pl.ds(c * _L, _L)] = ws
            il_v[pl.ds(c * _L, _L)] = v - ws

        def get_scalar(ref, b):
            chunk = ref[pl.ds((b // _L) * _L, _L)]
            return jnp.sum(jnp.where(lane16 == (b % _L), chunk, 0))

        def fetch(b, slot):
            us = pl.multiple_of(get_scalar(uws_v, b), 128)
            vs = pl.multiple_of(get_scalar(iws_v, b), 128)
            pltpu.async_copy(
                utab_t.at[:, pl.ds(us, 128)], uwin.at[slot], sems.at[slot])
            pltpu.async_copy(
                itab_t.at[:, pl.ds(vs, 128)], iwin.at[slot], sems.at[slot])

        def drain(slot):
            pltpu.make_async_copy(
                utab_t.at[:, pl.ds(0, 128)], uwin.at[slot], sems.at[slot]
            ).wait()
            pltpu.make_async_copy(
                itab_t.at[:, pl.ds(0, 128)], iwin.at[slot], sems.at[slot]
            ).wait()

        def compute(b, slot):
            ul = jnp.full((_L,), get_scalar(ul_v, b), jnp.int32)
            vl = jnp.full((_L,), get_scalar(il_v, b), jnp.int32)
            sl = jnp.full((_L,), slot, jnp.int32)
            u_lo = plsc.load_gather(uwin, [sl, lane16, ul])
            u_hi = plsc.load_gather(uwin, [sl, lane16 + _L, ul])
            i_lo = plsc.load_gather(iwin, [sl, lane16, vl])
            i_hi = plsc.load_gather(iwin, [sl, lane16 + _L, vl])
            s = u_lo * i_lo + u_hi * i_hi
            tot = jnp.sum(s)
            plsc.store_scatter(
                out_v, [jnp.full((_L,), b, jnp.int32)],
                jnp.full((_L,), tot, jnp.float32),
                mask=lane16 == 0)

        for s in range(_NBUF - 1):
            fetch(s, s)

        def body(b, carry):
            slot = lax.rem(b, _NBUF)
            fetch(b + (_NBUF - 1), lax.rem(b + _NBUF - 1, _NBUF))
            drain(slot)
            compute(b, slot)
            return carry

        lax.fori_loop(0, bpw - (_NBUF - 1), body, 0)

        for t in range(_NBUF - 1):
            b = bpw - (_NBUF - 1) + t
            drain(b % _NBUF)
            compute(b, b % _NBUF)

        pltpu.sync_copy(out_v, out_hbm.at[pl.ds(base, bpw)])

    return _k


def kernel(user_indices, item_indices, user_table, item_table):
    batch = user_indices.shape[0]
    k = _build(batch, user_table.shape[0])
    return k(user_indices.astype(jnp.int32), item_indices.astype(jnp.int32),
             user_table.T, item_table.T)
